# baseline (speedup 1.0000x reference)
import functools

import jax
import jax.numpy as jnp
import numpy as np
from jax import lax
from jax.experimental import pallas as pl
from jax.experimental.pallas import tpu as pltpu
from jax.experimental.pallas import tpu_sc as plsc

_U = np.uint32
_TINY = np.float32(1.1754943508222875e-38)
_ROT0 = (13, 15, 26, 6)
_ROT1 = (17, 29, 16, 24)
_NLN2 = np.float32(-0.6931471805599453)
_SQRT2 = np.float32(1.4142135623730951)
_LOG2P = (np.float32(-0.14745712), np.float32(0.23167853),
          np.float32(-0.24752942), np.float32(0.28740758),
          np.float32(-0.36037534), np.float32(0.4809117),
          np.float32(-0.7213499), np.float32(1.4426950408889634))


def _threefry_bits(x1):
    k1 = _U(0)
    k2 = _U(42)
    ks2 = _U(0 ^ 42 ^ 0x1BD11BDA)

    def rnd(x0, x1, r):
        x0 = x0 + x1
        x1 = (x1 << _U(r)) | (x1 >> _U(32 - r))
        x1 = x0 ^ x1
        return x0, x1

    x0 = x1
    x1 = ((x1 << _U(13)) | (x1 >> _U(19))) ^ x1
    for r in _ROT0[1:]:
        x0, x1 = rnd(x0, x1, r)
    x0 = x0 + k2
    x1 = x1 + (ks2 + _U(1))

    sched = ((_ROT1, ks2, k1, 2), (_ROT0, k1, k2, 3),
             (_ROT1, k2, ks2, 4), (_ROT0, ks2, k1, 5))
    for rots, a0, a1, c in sched:
        for r in rots:
            x0, x1 = rnd(x0, x1, r)
        x0 = x0 + a0
        x1 = x1 + (a1 + _U(c))
    return x0 ^ x1


def _bits_to_u(bits):
    f = lax.bitcast_convert_type((bits >> _U(9)) | _U(0x3F800000), jnp.float32)
    return jnp.maximum(f - jnp.float32(1.0), _TINY)


def _softlog2(x):
    xb = lax.bitcast_convert_type(x, jnp.uint32)
    e = ((xb >> _U(23)).astype(jnp.int32) - 127).astype(jnp.float32)
    m = lax.bitcast_convert_type((xb & _U(0x007FFFFF)) | _U(0x3F800000),
                                 jnp.float32)
    big = m >= _SQRT2
    m = jnp.where(big, m * np.float32(0.5), m)
    e = jnp.where(big, e + np.float32(1.0), e)
    r = m - np.float32(1.0)
    p = _LOG2P[0]
    for c in _LOG2P[1:]:
        p = p * r + c
    return e + r * p


def _soft_gumbel(bits):
    u = _bits_to_u(bits)
    t = _NLN2 * _softlog2(u)
    return _NLN2 * _softlog2(t)


def _tc_body(logits_ref, val_ref, loc_ref, base_iota, *, B, V, BLK, CH):
    j = pl.program_id(0)

    @pl.when(j == 0)
    def _init():
        val_ref[...] = jnp.full((B, 128), -jnp.inf, jnp.float32)
        loc_ref[...] = jnp.zeros((B, 128), jnp.int32)
        row = lax.broadcasted_iota(jnp.int32, (B, BLK), 0)
        col = lax.broadcasted_iota(jnp.int32, (B, BLK), 1)
        base_iota[...] = (row * V + col).astype(jnp.uint32)

    bv = val_ref[...]
    bl = loc_ref[...]
    off = (j * BLK + 42).astype(jnp.uint32)
    for c in range(BLK // CH):
        x = logits_ref[:, c * CH:(c + 1) * CH]
        bits = _threefry_bits(base_iota[:, c * CH:(c + 1) * CH] + off)
        u = _bits_to_u(bits)
        s = -jnp.log(-jnp.log(u)) + x
        nch = CH // 128
        for cc in range(nch):
            chunk = s[:, cc * 128:(cc + 1) * 128]
            upd = chunk > bv
            bv = jnp.where(upd, chunk, bv)
            bl = jnp.where(upd, j * (BLK // 128) + c * nch + cc, bl)
    val_ref[...] = bv
    loc_ref[...] = bl


def _tc_pass(logits, nbase):
    B, V = logits.shape
    BLK = 8192
    CH = 512
    return pl.pallas_call(
        functools.partial(_tc_body, B=B, V=V, BLK=BLK, CH=CH),
        grid=(nbase,),
        in_specs=[pl.BlockSpec((B, BLK), lambda j: (0, j))],
        out_specs=[pl.BlockSpec((B, 128), lambda j: (0, 0)),
                   pl.BlockSpec((B, 128), lambda j: (0, 0))],
        out_shape=[jax.ShapeDtypeStruct((B, 128), jnp.float32),
                   jax.ShapeDtypeStruct((B, 128), jnp.int32)],
        scratch_shapes=[pltpu.VMEM((B, BLK), jnp.uint32)],
    )(logits)


def _sc_pass(logits, V0, CHW, nchk):
    B, V = logits.shape
    Wsub = nchk * CHW
    nvec = CHW // 16
    mesh = plsc.VectorSubcoreMesh(core_axis_name="c", subcore_axis_name="s")

    @functools.partial(
        pl.kernel, mesh=mesh,
        out_type=[jax.ShapeDtypeStruct((B * 8 * 16,), jnp.float32),
                  jax.ShapeDtypeStruct((B * 8 * 16,), jnp.int32)],
        scratch_types=[pltpu.VMEM((8, CHW), jnp.float32),
                       pltpu.VMEM((16,), jnp.float32),
                       pltpu.VMEM((16,), jnp.int32),
                       pltpu.VMEM((128,), jnp.uint32),
                       pltpu.VMEM((8, 16), jnp.float32),
                       pltpu.VMEM((8, 16), jnp.int32)],
    )
    def sc_k(logits_hbm, val_hbm, col_hbm, chunk_v, val_v, col_v, bits_v,
             bvbuf, blbuf):
        wid = lax.axis_index("s") * 2 + lax.axis_index("c")
        rg = wid % 4
        cs = wid // 4
        iota16 = lax.iota(jnp.int32, 16)
        col0 = V0 + cs * Wsub
        for rr in range(8):
            bvbuf[rr, pl.ds(0, 16)] = jnp.full((16,), -jnp.inf, jnp.float32)
            blbuf[rr, pl.ds(0, 16)] = jnp.zeros((16,), jnp.int32)

        def chunk_body(ci, carry):
            pltpu.sync_copy(
                logits_hbm.at[pl.ds(rg * 8, 8), pl.ds(col0 + ci * CHW, CHW)],
                chunk_v)
            for rr in range(8):
                base = ((rg * 8 + rr) * V + col0 + 42).astype(jnp.int32)

                def lm_body(q, acc, rr=rr):
                    return jnp.maximum(acc, chunk_v[rr, pl.ds(q * 16, 16)])

                lmax16 = lax.fori_loop(
                    0, nvec, lm_body, jnp.full((16,), -jnp.inf, jnp.float32))
                bvmin16 = bvbuf[rr, pl.ds(0, 16)]
                for sh in (1, 2, 4, 8):
                    perm = iota16 ^ sh
                    lmax16 = jnp.maximum(lmax16, lmax16[perm])
                    bvmin16 = jnp.minimum(bvmin16, bvmin16[perm])
                c16 = bvmin16 - lmax16
                tv16 = jnp.exp(-jnp.exp(-c16))
                ti16 = (tv16 * np.float32(8388608.0)).astype(jnp.int32) - 16

                def batch_body(h, c2, rr=rr, base=base, ti16=ti16):
                    bm = jnp.zeros((16,), jnp.int32)
                    for t in range(8):
                        iv = h * 8 + t
                        i_vec = (iota16 + (base + ci * CHW + iv * 16)).astype(
                            jnp.uint32)
                        b = _threefry_bits(i_vec)
                        bits_v[pl.ds(t * 16, 16)] = b
                        bm = jnp.maximum(
                            bm, lax.bitcast_convert_type(b >> _U(9),
                                                         jnp.int32))
                    for sh in (1, 2, 4, 8):
                        bm = jnp.maximum(bm, bm[iota16 ^ sh])

                    @pl.when(bm[0] >= ti16[0])
                    def _hit(rr=rr, h=h):
                        bv = bvbuf[rr, pl.ds(0, 16)]
                        bl = blbuf[rr, pl.ds(0, 16)]
                        for t in range(8):
                            iv = h * 8 + t
                            b = bits_v[pl.ds(t * 16, 16)]
                            xl = chunk_v[rr, pl.ds(iv * 16, 16)]
                            s = _soft_gumbel(b) + xl
                            upd = s > bv
                            bv = jnp.where(upd, s, bv)
                            bl = jnp.where(upd, ci * nvec + iv, bl)
                        bvbuf[rr, pl.ds(0, 16)] = bv
                        blbuf[rr, pl.ds(0, 16)] = bl

                    return c2

                lax.fori_loop(0, nvec // 8, batch_body, 0)
            return carry

        lax.fori_loop(0, nchk, chunk_body, 0)
        for rr in range(8):
            val_v[...] = bvbuf[rr, pl.ds(0, 16)]
            col_v[...] = col0 + blbuf[rr, pl.ds(0, 16)] * 16 + iota16
            slot = ((rg * 8 + rr) * 8 + cs) * 16
            pltpu.sync_copy(val_v, val_hbm.at[pl.ds(slot, 16)])
            pltpu.sync_copy(col_v, col_hbm.at[pl.ds(slot, 16)])

    val1d, col1d = sc_k(logits)
    return val1d.reshape(B, 128), col1d.reshape(B, 128)


def _merge_body(logits_ref, tcv_ref, tcl_ref, scv_ref, scc_ref, out_ref, *,
                B, V, ctail, TAILP):
    bv = tcv_ref[...]
    bl = tcl_ref[...]
    if ctail < V:
        x = logits_ref[...]
        row = lax.broadcasted_iota(jnp.int32, (B, TAILP), 0)
        col = lax.broadcasted_iota(jnp.int32, (B, TAILP), 1)
        bits = _threefry_bits((row * V + col + (ctail + 42)).astype(
            jnp.uint32))
        u = _bits_to_u(bits)
        s = -jnp.log(-jnp.log(u)) + x
        valid = col < (V - ctail)
        for cc in range(TAILP // 128):
            chunk = s[:, cc * 128:(cc + 1) * 128]
            upd = (chunk > bv) & valid[:, cc * 128:(cc + 1) * 128]
            bv = jnp.where(upd, chunk, bv)
            bl = jnp.where(upd, ctail // 128 + cc, bl)
    imax = jnp.int32(2**31 - 1)
    m_tc = jnp.max(bv, axis=1, keepdims=True)
    lane = lax.broadcasted_iota(jnp.int32, (B, 128), 1)
    cand = jnp.where(bv == m_tc, bl * 128 + lane, imax)
    c_tc = jnp.min(cand, axis=1, keepdims=True)
    scv = scv_ref[...]
    m_sc = jnp.max(scv, axis=1, keepdims=True)
    cand_sc = jnp.where(scv == m_sc, scc_ref[...], imax)
    c_sc = jnp.min(cand_sc, axis=1, keepdims=True)
    best = jnp.where(m_sc > m_tc, c_sc, c_tc)
    out_ref[...] = jnp.broadcast_to(best, (B, 128))


def _merge(logits, tcv, tcl, scv, scc, ctail):
    B, V = logits.shape
    TAILP = 1024 if V - ctail <= 1024 else 8192
    return pl.pallas_call(
        functools.partial(_merge_body, B=B, V=V, ctail=ctail, TAILP=TAILP),
        grid=(1,),
        in_specs=[pl.BlockSpec((B, TAILP),
                               lambda j: (0, ctail // TAILP if ctail < V
                                          else 0)),
                  pl.BlockSpec((B, 128), lambda j: (0, 0)),
                  pl.BlockSpec((B, 128), lambda j: (0, 0)),
                  pl.BlockSpec((B, 128), lambda j: (0, 0)),
                  pl.BlockSpec((B, 128), lambda j: (0, 0))],
        out_specs=pl.BlockSpec((B, 128), lambda j: (0, 0)),
        out_shape=jax.ShapeDtypeStruct((B, 128), jnp.int32),
    )(logits, tcv, tcl, scv, scc)


def kernel(logits):
    B, V = logits.shape
    BLK = 8192
    CHW = 2048
    NCHK = 12
    W_SC = 8 * NCHK * CHW
    nb = V // BLK
    nbase = nb - W_SC // BLK
    if B == 32 and nbase > 0:
        tcv, tcl = _tc_pass(logits, nbase)
        scv, scc = _sc_pass(logits, nbase * BLK, CHW, NCHK)
        out = _merge(logits, tcv, tcl, scv, scc, nb * BLK)
    else:
        if nb > 0:
            tcv, tcl = _tc_pass(logits, nb)
        else:
            tcv = jnp.full((B, 128), -jnp.inf, jnp.float32)
            tcl = jnp.zeros((B, 128), jnp.int32)
        scv = jnp.full((B, 128), -jnp.inf, jnp.float32)
        scc = jnp.zeros((B, 128), jnp.int32)
        out = _merge(logits, tcv, tcl, scv, scc, nb * BLK)
    return out[:, 0].astype(jnp.int64)


# speedup vs baseline: 1.2144x; 1.2144x over previous
import functools

import jax
import jax.numpy as jnp
import numpy as np
from jax import lax
from jax.experimental import pallas as pl
from jax.experimental.pallas import tpu as pltpu
from jax.experimental.pallas import tpu_sc as plsc

_U = np.uint32
_TINY = np.float32(1.1754943508222875e-38)
_ROT0 = (13, 15, 26, 6)
_ROT1 = (17, 29, 16, 24)
_NLN2 = np.float32(-0.6931471805599453)
_SQRT2 = np.float32(1.4142135623730951)
_LOG2P = (np.float32(-0.14745712), np.float32(0.23167853),
          np.float32(-0.24752942), np.float32(0.28740758),
          np.float32(-0.36037534), np.float32(0.4809117),
          np.float32(-0.7213499), np.float32(1.4426950408889634))


def _threefry_bits(x1):
    k1 = _U(0)
    k2 = _U(42)
    ks2 = _U(0 ^ 42 ^ 0x1BD11BDA)

    def rnd(x0, x1, r):
        x0 = x0 + x1
        x1 = (x1 << _U(r)) | (x1 >> _U(32 - r))
        x1 = x0 ^ x1
        return x0, x1

    x0 = x1
    x1 = ((x1 << _U(13)) | (x1 >> _U(19))) ^ x1
    for r in _ROT0[1:]:
        x0, x1 = rnd(x0, x1, r)
    x0 = x0 + k2
    x1 = x1 + (ks2 + _U(1))

    sched = ((_ROT1, ks2, k1, 2), (_ROT0, k1, k2, 3),
             (_ROT1, k2, ks2, 4), (_ROT0, ks2, k1, 5))
    for rots, a0, a1, c in sched:
        for r in rots:
            x0, x1 = rnd(x0, x1, r)
        x0 = x0 + a0
        x1 = x1 + (a1 + _U(c))
    return x0 ^ x1


def _bits_to_u(bits):
    f = lax.bitcast_convert_type((bits >> _U(9)) | _U(0x3F800000), jnp.float32)
    return jnp.maximum(f - jnp.float32(1.0), _TINY)


def _softlog2(x):
    xb = lax.bitcast_convert_type(x, jnp.uint32)
    e = ((xb >> _U(23)).astype(jnp.int32) - 127).astype(jnp.float32)
    m = lax.bitcast_convert_type((xb & _U(0x007FFFFF)) | _U(0x3F800000),
                                 jnp.float32)
    big = m >= _SQRT2
    m = jnp.where(big, m * np.float32(0.5), m)
    e = jnp.where(big, e + np.float32(1.0), e)
    r = m - np.float32(1.0)
    p = _LOG2P[0]
    for c in _LOG2P[1:]:
        p = p * r + c
    return e + r * p


def _soft_gumbel(bits):
    u = _bits_to_u(bits)
    t = _NLN2 * _softlog2(u)
    return _NLN2 * _softlog2(t)


def _tc_body(logits_ref, val_ref, loc_ref, base_iota, *, B, V, BLK, CH):
    j = pl.program_id(0)

    @pl.when(j == 0)
    def _init():
        val_ref[...] = jnp.full((B, 128), -jnp.inf, jnp.float32)
        loc_ref[...] = jnp.zeros((B, 128), jnp.int32)
        row = lax.broadcasted_iota(jnp.int32, (B, BLK), 0)
        col = lax.broadcasted_iota(jnp.int32, (B, BLK), 1)
        base_iota[...] = (row * V + col).astype(jnp.uint32)

    bv = val_ref[...]
    bl = loc_ref[...]
    off = (j * BLK + 42).astype(jnp.uint32)
    for c in range(BLK // CH):
        x = logits_ref[:, c * CH:(c + 1) * CH]
        bits = _threefry_bits(base_iota[:, c * CH:(c + 1) * CH] + off)
        u = _bits_to_u(bits)
        s = -jnp.log(-jnp.log(u)) + x
        nch = CH // 128
        for cc in range(nch):
            chunk = s[:, cc * 128:(cc + 1) * 128]
            upd = chunk > bv
            bv = jnp.where(upd, chunk, bv)
            bl = jnp.where(upd, j * (BLK // 128) + c * nch + cc, bl)
    val_ref[...] = bv
    loc_ref[...] = bl


def _tc_pass(logits, nbase):
    B, V = logits.shape
    BLK = 8192
    CH = 512
    return pl.pallas_call(
        functools.partial(_tc_body, B=B, V=V, BLK=BLK, CH=CH),
        grid=(nbase,),
        in_specs=[pl.BlockSpec((B, BLK), lambda j: (0, j))],
        out_specs=[pl.BlockSpec((B, 128), lambda j: (0, 0)),
                   pl.BlockSpec((B, 128), lambda j: (0, 0))],
        out_shape=[jax.ShapeDtypeStruct((B, 128), jnp.float32),
                   jax.ShapeDtypeStruct((B, 128), jnp.int32)],
        scratch_shapes=[pltpu.VMEM((B, BLK), jnp.uint32)],
    )(logits)


def _sc_pass(logits, V0, CHW, nchk):
    B, V = logits.shape
    Wsub = nchk * CHW
    nvec = CHW // 16
    mesh = plsc.VectorSubcoreMesh(core_axis_name="c", subcore_axis_name="s")

    @functools.partial(
        pl.kernel, mesh=mesh,
        out_type=[jax.ShapeDtypeStruct((B * 8 * 16,), jnp.float32),
                  jax.ShapeDtypeStruct((B * 8 * 16,), jnp.int32)],
        scratch_types=[pltpu.VMEM((8, CHW), jnp.float32),
                       pltpu.VMEM((16,), jnp.float32),
                       pltpu.VMEM((16,), jnp.int32),
                       pltpu.VMEM((CHW,), jnp.uint32),
                       pltpu.VMEM((8, 16), jnp.float32),
                       pltpu.VMEM((8, 16), jnp.int32)],
    )
    def sc_k(logits_hbm, val_hbm, col_hbm, chunk_v, val_v, col_v, bits_v,
             bvbuf, blbuf):
        wid = lax.axis_index("s") * 2 + lax.axis_index("c")
        rg = wid % 4
        cs = wid // 4
        iota16 = lax.iota(jnp.int32, 16)
        col0 = V0 + cs * Wsub
        for rr in range(8):
            bvbuf[rr, pl.ds(0, 16)] = jnp.full((16,), -jnp.inf, jnp.float32)
            blbuf[rr, pl.ds(0, 16)] = jnp.zeros((16,), jnp.int32)

        def chunk_body(ci, carry):
            pltpu.sync_copy(
                logits_hbm.at[pl.ds(rg * 8, 8), pl.ds(col0 + ci * CHW, CHW)],
                chunk_v)
            for rr in range(8):
                base = ((rg * 8 + rr) * V + col0 + 42).astype(jnp.int32)

                def pass1(q, c2, rr=rr, base=base):
                    bm, lmax = c2
                    i_vec = (iota16 + (base + ci * CHW + q * 16)).astype(
                        jnp.uint32)
                    b = _threefry_bits(i_vec)
                    bits_v[pl.ds(q * 16, 16)] = b
                    bm = jnp.maximum(
                        bm, lax.bitcast_convert_type(b >> _U(9), jnp.int32))
                    lmax = jnp.maximum(lmax, chunk_v[rr, pl.ds(q * 16, 16)])
                    return bm, lmax

                bm16, lmax16 = lax.fori_loop(
                    0, nvec, pass1,
                    (jnp.zeros((16,), jnp.int32),
                     jnp.full((16,), -jnp.inf, jnp.float32)))
                bvmin16 = bvbuf[rr, pl.ds(0, 16)]
                for sh in (1, 2, 4, 8):
                    perm = iota16 ^ sh
                    lmax16 = jnp.maximum(lmax16, lmax16[perm])
                    bvmin16 = jnp.minimum(bvmin16, bvmin16[perm])
                    bm16 = jnp.maximum(bm16, bm16[perm])
                tv16 = jnp.exp(-jnp.exp(lmax16 - bvmin16))
                ti16 = (tv16 * np.float32(8388608.0)).astype(jnp.int32) - 16

                @pl.when(bm16[0] >= ti16[0])
                def _hit(rr=rr, ci=ci):
                    def pass2(q, c3, rr=rr):
                        bv, bl = c3
                        b = bits_v[pl.ds(q * 16, 16)]
                        xl = chunk_v[rr, pl.ds(q * 16, 16)]
                        s = _soft_gumbel(b) + xl
                        upd = s > bv
                        bv = jnp.where(upd, s, bv)
                        bl = jnp.where(upd, ci * nvec + q, bl)
                        return bv, bl

                    bv, bl = lax.fori_loop(
                        0, nvec, pass2,
                        (bvbuf[rr, pl.ds(0, 16)], blbuf[rr, pl.ds(0, 16)]))
                    bvbuf[rr, pl.ds(0, 16)] = bv
                    blbuf[rr, pl.ds(0, 16)] = bl
            return carry

        lax.fori_loop(0, nchk, chunk_body, 0)
        for rr in range(8):
            val_v[...] = bvbuf[rr, pl.ds(0, 16)]
            col_v[...] = col0 + blbuf[rr, pl.ds(0, 16)] * 16 + iota16
            slot = ((rg * 8 + rr) * 8 + cs) * 16
            pltpu.sync_copy(val_v, val_hbm.at[pl.ds(slot, 16)])
            pltpu.sync_copy(col_v, col_hbm.at[pl.ds(slot, 16)])

    val1d, col1d = sc_k(logits)
    return val1d.reshape(B, 128), col1d.reshape(B, 128)


def _merge_body(logits_ref, tcv_ref, tcl_ref, scv_ref, scc_ref, out_ref, *,
                B, V, ctail, TAILP):
    bv = tcv_ref[...]
    bl = tcl_ref[...]
    if ctail < V:
        x = logits_ref[...]
        row = lax.broadcasted_iota(jnp.int32, (B, TAILP), 0)
        col = lax.broadcasted_iota(jnp.int32, (B, TAILP), 1)
        bits = _threefry_bits((row * V + col + (ctail + 42)).astype(
            jnp.uint32))
        u = _bits_to_u(bits)
        s = -jnp.log(-jnp.log(u)) + x
        valid = col < (V - ctail)
        for cc in range(TAILP // 128):
            chunk = s[:, cc * 128:(cc + 1) * 128]
            upd = (chunk > bv) & valid[:, cc * 128:(cc + 1) * 128]
            bv = jnp.where(upd, chunk, bv)
            bl = jnp.where(upd, ctail // 128 + cc, bl)
    imax = jnp.int32(2**31 - 1)
    m_tc = jnp.max(bv, axis=1, keepdims=True)
    lane = lax.broadcasted_iota(jnp.int32, (B, 128), 1)
    cand = jnp.where(bv == m_tc, bl * 128 + lane, imax)
    c_tc = jnp.min(cand, axis=1, keepdims=True)
    scv = scv_ref[...]
    m_sc = jnp.max(scv, axis=1, keepdims=True)
    cand_sc = jnp.where(scv == m_sc, scc_ref[...], imax)
    c_sc = jnp.min(cand_sc, axis=1, keepdims=True)
    best = jnp.where(m_sc > m_tc, c_sc, c_tc)
    out_ref[...] = jnp.broadcast_to(best, (B, 128))


def _merge(logits, tcv, tcl, scv, scc, ctail):
    B, V = logits.shape
    TAILP = 1024 if V - ctail <= 1024 else 8192
    return pl.pallas_call(
        functools.partial(_merge_body, B=B, V=V, ctail=ctail, TAILP=TAILP),
        grid=(1,),
        in_specs=[pl.BlockSpec((B, TAILP),
                               lambda j: (0, ctail // TAILP if ctail < V
                                          else 0)),
                  pl.BlockSpec((B, 128), lambda j: (0, 0)),
                  pl.BlockSpec((B, 128), lambda j: (0, 0)),
                  pl.BlockSpec((B, 128), lambda j: (0, 0)),
                  pl.BlockSpec((B, 128), lambda j: (0, 0))],
        out_specs=pl.BlockSpec((B, 128), lambda j: (0, 0)),
        out_shape=jax.ShapeDtypeStruct((B, 128), jnp.int32),
    )(logits, tcv, tcl, scv, scc)


def kernel(logits):
    B, V = logits.shape
    BLK = 8192
    CHW = 2048
    NCHK = 12
    W_SC = 8 * NCHK * CHW
    nb = V // BLK
    nbase = nb - W_SC // BLK
    if B == 32 and nbase > 0:
        tcv, tcl = _tc_pass(logits, nbase)
        scv, scc = _sc_pass(logits, nbase * BLK, CHW, NCHK)
        out = _merge(logits, tcv, tcl, scv, scc, nb * BLK)
    else:
        if nb > 0:
            tcv, tcl = _tc_pass(logits, nb)
        else:
            tcv = jnp.full((B, 128), -jnp.inf, jnp.float32)
            tcl = jnp.zeros((B, 128), jnp.int32)
        scv = jnp.full((B, 128), -jnp.inf, jnp.float32)
        scc = jnp.zeros((B, 128), jnp.int32)
        out = _merge(logits, tcv, tcl, scv, scc, nb * BLK)
    return out[:, 0].astype(jnp.int64)


# speedup vs baseline: 1.3315x; 1.0964x over previous
import functools

import jax
import jax.numpy as jnp
import numpy as np
from jax import lax
from jax.experimental import pallas as pl
from jax.experimental.pallas import tpu as pltpu
from jax.experimental.pallas import tpu_sc as plsc

_U = np.uint32
_TINY = np.float32(1.1754943508222875e-38)
_ROT0 = (13, 15, 26, 6)
_ROT1 = (17, 29, 16, 24)
_NLN2 = np.float32(-0.6931471805599453)
_SQRT2 = np.float32(1.4142135623730951)
_LOG2P = (np.float32(-0.14745712), np.float32(0.23167853),
          np.float32(-0.24752942), np.float32(0.28740758),
          np.float32(-0.36037534), np.float32(0.4809117),
          np.float32(-0.7213499), np.float32(1.4426950408889634))


def _threefry_bits(x1):
    k1 = _U(0)
    k2 = _U(42)
    ks2 = _U(0 ^ 42 ^ 0x1BD11BDA)

    def rnd(x0, x1, r):
        x0 = x0 + x1
        x1 = (x1 << _U(r)) | (x1 >> _U(32 - r))
        x1 = x0 ^ x1
        return x0, x1

    x0 = x1
    x1 = ((x1 << _U(13)) | (x1 >> _U(19))) ^ x1
    for r in _ROT0[1:]:
        x0, x1 = rnd(x0, x1, r)
    x0 = x0 + k2
    x1 = x1 + (ks2 + _U(1))

    sched = ((_ROT1, ks2, k1, 2), (_ROT0, k1, k2, 3),
             (_ROT1, k2, ks2, 4), (_ROT0, ks2, k1, 5))
    for rots, a0, a1, c in sched:
        for r in rots:
            x0, x1 = rnd(x0, x1, r)
        x0 = x0 + a0
        x1 = x1 + (a1 + _U(c))
    return x0 ^ x1


def _bits_to_u(bits):
    f = lax.bitcast_convert_type((bits >> _U(9)) | _U(0x3F800000), jnp.float32)
    return jnp.maximum(f - jnp.float32(1.0), _TINY)


def _softlog2(x):
    xb = lax.bitcast_convert_type(x, jnp.uint32)
    e = ((xb >> _U(23)).astype(jnp.int32) - 127).astype(jnp.float32)
    m = lax.bitcast_convert_type((xb & _U(0x007FFFFF)) | _U(0x3F800000),
                                 jnp.float32)
    big = m >= _SQRT2
    m = jnp.where(big, m * np.float32(0.5), m)
    e = jnp.where(big, e + np.float32(1.0), e)
    r = m - np.float32(1.0)
    p = _LOG2P[0]
    for c in _LOG2P[1:]:
        p = p * r + c
    return e + r * p


def _soft_gumbel(bits):
    u = _bits_to_u(bits)
    t = _NLN2 * _softlog2(u)
    return _NLN2 * _softlog2(t)


def _tc_body(logits_ref, val_ref, loc_ref, base_iota, *, B, V, BLK, CH):
    j = pl.program_id(0)

    @pl.when(j == 0)
    def _init():
        val_ref[...] = jnp.full((B, 128), -jnp.inf, jnp.float32)
        loc_ref[...] = jnp.zeros((B, 128), jnp.int32)
        row = lax.broadcasted_iota(jnp.int32, (B, BLK), 0)
        col = lax.broadcasted_iota(jnp.int32, (B, BLK), 1)
        base_iota[...] = (row * V + col).astype(jnp.uint32)

    bv = val_ref[...]
    bl = loc_ref[...]
    off = (j * BLK + 42).astype(jnp.uint32)
    for c in range(BLK // CH):
        x = logits_ref[:, c * CH:(c + 1) * CH]
        bits = _threefry_bits(base_iota[:, c * CH:(c + 1) * CH] + off)
        u = _bits_to_u(bits)
        s = -jnp.log(-jnp.log(u)) + x
        nch = CH // 128
        for cc in range(nch):
            chunk = s[:, cc * 128:(cc + 1) * 128]
            upd = chunk > bv
            bv = jnp.where(upd, chunk, bv)
            bl = jnp.where(upd, j * (BLK // 128) + c * nch + cc, bl)
    val_ref[...] = bv
    loc_ref[...] = bl


def _tc_pass(logits, nbase):
    B, V = logits.shape
    BLK = 8192
    CH = 512
    return pl.pallas_call(
        functools.partial(_tc_body, B=B, V=V, BLK=BLK, CH=CH),
        grid=(nbase,),
        in_specs=[pl.BlockSpec((B, BLK), lambda j: (0, j))],
        out_specs=[pl.BlockSpec((B, 128), lambda j: (0, 0)),
                   pl.BlockSpec((B, 128), lambda j: (0, 0))],
        out_shape=[jax.ShapeDtypeStruct((B, 128), jnp.float32),
                   jax.ShapeDtypeStruct((B, 128), jnp.int32)],
        scratch_shapes=[pltpu.VMEM((B, BLK), jnp.uint32)],
    )(logits)


def _sc_pass(logits, V0, CHW, nchk):
    B, V = logits.shape
    Wsub = nchk * CHW
    nvec = CHW // 16
    mesh = plsc.VectorSubcoreMesh(core_axis_name="c", subcore_axis_name="s")

    @functools.partial(
        pl.kernel, mesh=mesh,
        out_type=[jax.ShapeDtypeStruct((B * 8 * 16,), jnp.float32),
                  jax.ShapeDtypeStruct((B * 8 * 16,), jnp.int32)],
        scratch_types=[pltpu.VMEM((8, CHW), jnp.float32),
                       pltpu.VMEM((16,), jnp.float32),
                       pltpu.VMEM((16,), jnp.int32),
                       pltpu.VMEM((CHW,), jnp.uint32),
                       pltpu.VMEM((8, 16), jnp.float32),
                       pltpu.VMEM((8, 16), jnp.int32)],
    )
    def sc_k(logits_hbm, val_hbm, col_hbm, chunk_v, val_v, col_v, bits_v,
             bvbuf, blbuf):
        wid = lax.axis_index("s") * 2 + lax.axis_index("c")
        rg = wid % 4
        cs = wid // 4
        iota16 = lax.iota(jnp.int32, 16)
        col0 = V0 + cs * Wsub
        for rr in range(8):
            bvbuf[rr, pl.ds(0, 16)] = jnp.full((16,), -jnp.inf, jnp.float32)
            blbuf[rr, pl.ds(0, 16)] = jnp.zeros((16,), jnp.int32)

        def chunk_body(ci, carry):
            pltpu.sync_copy(
                logits_hbm.at[pl.ds(rg * 8, 8), pl.ds(col0 + ci * CHW, CHW)],
                chunk_v)
            for rr in range(8):
                base = ((rg * 8 + rr) * V + col0 + 42).astype(jnp.int32)

                def vec_body(q, c2, rr=rr, base=base):
                    bv, bl = c2
                    xl = chunk_v[rr, pl.ds(q * 16, 16)]
                    i_vec = (iota16 + (base + ci * CHW + q * 16)).astype(
                        jnp.uint32)
                    s = _soft_gumbel(_threefry_bits(i_vec)) + xl
                    upd = s > bv
                    bv = jnp.where(upd, s, bv)
                    bl = jnp.where(upd, ci * nvec + q, bl)
                    return bv, bl

                bv, bl = lax.fori_loop(
                    0, nvec, vec_body,
                    (bvbuf[rr, pl.ds(0, 16)], blbuf[rr, pl.ds(0, 16)]))
                bvbuf[rr, pl.ds(0, 16)] = bv
                blbuf[rr, pl.ds(0, 16)] = bl
            return carry

        lax.fori_loop(0, nchk, chunk_body, 0)
        for rr in range(8):
            val_v[...] = bvbuf[rr, pl.ds(0, 16)]
            col_v[...] = col0 + blbuf[rr, pl.ds(0, 16)] * 16 + iota16
            slot = ((rg * 8 + rr) * 8 + cs) * 16
            pltpu.sync_copy(val_v, val_hbm.at[pl.ds(slot, 16)])
            pltpu.sync_copy(col_v, col_hbm.at[pl.ds(slot, 16)])

    val1d, col1d = sc_k(logits)
    return val1d.reshape(B, 128), col1d.reshape(B, 128)


def _merge_body(logits_ref, tcv_ref, tcl_ref, scv_ref, scc_ref, out_ref, *,
                B, V, ctail, TAILP):
    bv = tcv_ref[...]
    bl = tcl_ref[...]
    if ctail < V:
        x = logits_ref[...]
        row = lax.broadcasted_iota(jnp.int32, (B, TAILP), 0)
        col = lax.broadcasted_iota(jnp.int32, (B, TAILP), 1)
        bits = _threefry_bits((row * V + col + (ctail + 42)).astype(
            jnp.uint32))
        u = _bits_to_u(bits)
        s = -jnp.log(-jnp.log(u)) + x
        valid = col < (V - ctail)
        for cc in range(TAILP // 128):
            chunk = s[:, cc * 128:(cc + 1) * 128]
            upd = (chunk > bv) & valid[:, cc * 128:(cc + 1) * 128]
            bv = jnp.where(upd, chunk, bv)
            bl = jnp.where(upd, ctail // 128 + cc, bl)
    imax = jnp.int32(2**31 - 1)
    m_tc = jnp.max(bv, axis=1, keepdims=True)
    lane = lax.broadcasted_iota(jnp.int32, (B, 128), 1)
    cand = jnp.where(bv == m_tc, bl * 128 + lane, imax)
    c_tc = jnp.min(cand, axis=1, keepdims=True)
    scv = scv_ref[...]
    m_sc = jnp.max(scv, axis=1, keepdims=True)
    cand_sc = jnp.where(scv == m_sc, scc_ref[...], imax)
    c_sc = jnp.min(cand_sc, axis=1, keepdims=True)
    best = jnp.where(m_sc > m_tc, c_sc, c_tc)
    out_ref[...] = jnp.broadcast_to(best, (B, 128))


def _merge(logits, tcv, tcl, scv, scc, ctail):
    B, V = logits.shape
    TAILP = 1024 if V - ctail <= 1024 else 8192
    return pl.pallas_call(
        functools.partial(_merge_body, B=B, V=V, ctail=ctail, TAILP=TAILP),
        grid=(1,),
        in_specs=[pl.BlockSpec((B, TAILP),
                               lambda j: (0, ctail // TAILP if ctail < V
                                          else 0)),
                  pl.BlockSpec((B, 128), lambda j: (0, 0)),
                  pl.BlockSpec((B, 128), lambda j: (0, 0)),
                  pl.BlockSpec((B, 128), lambda j: (0, 0)),
                  pl.BlockSpec((B, 128), lambda j: (0, 0))],
        out_specs=pl.BlockSpec((B, 128), lambda j: (0, 0)),
        out_shape=jax.ShapeDtypeStruct((B, 128), jnp.int32),
    )(logits, tcv, tcl, scv, scc)


def kernel(logits):
    B, V = logits.shape
    BLK = 8192
    CHW = 2048
    NCHK = 11
    W_SC = 8 * NCHK * CHW
    nb = V // BLK
    nbase = nb - W_SC // BLK
    if B == 32 and nbase > 0:
        tcv, tcl = _tc_pass(logits, nbase)
        scv, scc = _sc_pass(logits, nbase * BLK, CHW, NCHK)
        out = _merge(logits, tcv, tcl, scv, scc, nb * BLK)
    else:
        if nb > 0:
            tcv, tcl = _tc_pass(logits, nb)
        else:
            tcv = jnp.full((B, 128), -jnp.inf, jnp.float32)
            tcl = jnp.zeros((B, 128), jnp.int32)
        scv = jnp.full((B, 128), -jnp.inf, jnp.float32)
        scc = jnp.zeros((B, 128), jnp.int32)
        out = _merge(logits, tcv, tcl, scv, scc, nb * BLK)
    return out[:, 0].astype(jnp.int64)
